# baseline (speedup 1.0000x reference)
import functools

import jax
import jax.numpy as jnp
from jax import lax
from jax.experimental import pallas as pl
from jax.experimental.pallas import tpu as pltpu
from jax.experimental.pallas import tpu_sc as plsc

BATCH = 16384
FEAT = 64
NUM_CLASSES = 100000

NC = 2
NS = 16
NW = NC * NS
BPW = BATCH // NW
TBL = 100096
TBL_PER_TILE = TBL // NS


def _sc_body(labels_hbm, feature_hbm, centers_hbm,
             sumsq_hbm, num_hbm,
             idx_v, idx2_v, cidx_v, rows_v, feat_v, num_v, sumsq_v,
             zeros_v, ones_v, table, rows_sem, feat_sem):
    c = lax.axis_index("c")
    s = lax.axis_index("s")
    w = c * NS + s

    pltpu.sync_copy(labels_hbm.at[pl.ds(w * 4, 4)], idx_v)

    for j in range(4):
        for k in range(8):
            v = idx_v[j, pl.ds(k * 16, 16)]
            idx2_v[j, pl.ds(k * 16, 16)] = lax.shift_right_logical(v, 1)

    row_cps = [
        pltpu.async_copy(centers_hbm.at[idx2_v.at[j]],
                         rows_v.at[pl.ds(j * 128, 128)], rows_sem)
        for j in range(4)
    ]
    feat_cp = pltpu.async_copy(feature_hbm.at[pl.ds(w * 256, 256)],
                               feat_v, feat_sem)

    def _zbody(k, _):
        zeros_v[pl.ds(k * 16, 16)] = jnp.zeros((16,), jnp.float32)
        return ()
    lax.fori_loop(0, TBL_PER_TILE // 16, _zbody, ())
    for k in range(8):
        ones_v[pl.ds(k * 16, 16)] = jnp.ones((16,), jnp.float32)

    pltpu.sync_copy(zeros_v, table.at[pl.ds(s * TBL_PER_TILE, TBL_PER_TILE)])
    plsc.subcore_barrier()
    pltpu.sync_copy(labels_hbm.at[pl.ds(s * 8, 8)], cidx_v)
    for j in range(8):
        pltpu.sync_copy(ones_v, table.at[cidx_v.at[j]], add=True)
    plsc.subcore_barrier()

    for j in range(4):
        pltpu.sync_copy(table.at[idx_v.at[j]], num_v.at[pl.ds(j * 128, 128)])

    for cp in row_cps:
        cp.wait()
    feat_cp.wait()

    lane = lax.iota(jnp.int32, 16)

    def _gbody(g, _):
        labvec = idx_v[g >> 3, pl.ds((g & 7) * 16, 16)]
        acc = jnp.zeros((16,), jnp.float32)
        for j in range(16):
            i = g * 16 + j
            lab = labvec[j]
            odd = lax.bitwise_and(lab, 1) == 1
            frow = g * 8 + (j // 2)
            fbase = (j % 2) * 64
            t = jnp.zeros((16,), jnp.float32)
            for ch in range(4):
                f = feat_v[frow, pl.ds(fbase + ch * 16, 16)]
                ce = rows_v[i, pl.ds(ch * 16, 16)]
                co = rows_v[i, pl.ds(64 + ch * 16, 16)]
                d = f - jnp.where(odd, co, ce)
                t = t + d * d
            acc = jnp.where(lane == j, jnp.sum(t), acc)
        sumsq_v[pl.ds(g * 16, 16)] = acc
        return ()
    lax.fori_loop(0, BPW // 16, _gbody, ())

    pltpu.sync_copy(sumsq_v, sumsq_hbm.at[pl.ds(w * BPW, BPW)])
    pltpu.sync_copy(num_v, num_hbm.at[pl.ds(w * BPW, BPW)])


@jax.jit
def _sc_stage(labels2d, feature2d, centers2d):
    mesh = plsc.VectorSubcoreMesh(core_axis_name="c", subcore_axis_name="s")
    fn = pl.kernel(
        _sc_body,
        out_type=(
            jax.ShapeDtypeStruct((BATCH,), jnp.float32),
            jax.ShapeDtypeStruct((BATCH,), jnp.float32),
        ),
        mesh=mesh,
        compiler_params=pltpu.CompilerParams(
            needs_layout_passes=False, use_tc_tiling_on_sc=True),
        scratch_types=[
            pltpu.VMEM((4, 128), jnp.int32),
            pltpu.VMEM((4, 128), jnp.int32),
            pltpu.VMEM((8, 128), jnp.int32),
            pltpu.VMEM((BPW, 128), jnp.float32),
            pltpu.VMEM((256, 128), jnp.float32),
            pltpu.VMEM((BPW,), jnp.float32),
            pltpu.VMEM((BPW,), jnp.float32),
            pltpu.VMEM((TBL_PER_TILE,), jnp.float32),
            pltpu.VMEM((128,), jnp.float32),
            pltpu.VMEM_SHARED((TBL,), jnp.float32),
            pltpu.SemaphoreType.DMA,
            pltpu.SemaphoreType.DMA,
        ],
    )
    return fn(labels2d, feature2d, centers2d)


def _loss_body(sumsq_ref, num_ref, out_ref):
    dist = jnp.sqrt(sumsq_ref[...])
    loss = jnp.sum(dist / num_ref[...]) * (1.0 / BATCH)
    out_ref[...] = loss.reshape(1, 1)


@jax.jit
def _tc_stage(sumsq, num):
    out = pl.pallas_call(
        _loss_body,
        out_shape=jax.ShapeDtypeStruct((1, 1), jnp.float32),
    )(sumsq.reshape(128, 128), num.reshape(128, 128))
    return out[0, 0]


def kernel(feature, label, centers):
    label = jnp.asarray(label, jnp.int32)
    labels2d = label.reshape(128, 128)
    feature2d = feature.reshape(8192, 128)
    centers2d = centers.reshape(50000, 128)
    sumsq, num = _sc_stage(labels2d, feature2d, centers2d)
    return _tc_stage(sumsq, num)


# speedup vs baseline: 1.2289x; 1.2289x over previous
import functools

import jax
import jax.numpy as jnp
from jax import lax
from jax.experimental import pallas as pl
from jax.experimental.pallas import tpu as pltpu
from jax.experimental.pallas import tpu_sc as plsc

BATCH = 16384
FEAT = 64
NUM_CLASSES = 100000

NC = 2
NS = 16
NW = NC * NS
BPW = BATCH // NW
HLF = 50048
HTBL = 50176
HPT = HTBL // NS
DUMMY = HLF
CHUNK = 2048
NCHUNK = BATCH // CHUNK


def _sc_body(labels_hbm, featT_hbm, centersT_hbm,
             parts_hbm, num_hbm,
             dimrow_v, partial_v, lab_v, featd_v, num_v, ones_v,
             table, dim_sem, ch_sems):
    c = lax.axis_index("c")
    s = lax.axis_index("s")
    w = c * NS + s

    dim0 = c * 32 + s
    dim_cps = [pltpu.async_copy(centersT_hbm.at[dim0], dimrow_v, dim_sem)]

    def _zbody(k, _):
        partial_v[pl.ds(k * 16, 16)] = jnp.zeros((16,), jnp.float32)
        return ()
    lax.fori_loop(0, HPT // 16, _zbody, ())
    for k in range(8):
        ones_v[pl.ds(k * 16, 16)] = jnp.ones((16,), jnp.float32)

    for h in range(2):
        lo = h * HLF
        pltpu.sync_copy(partial_v.at[pl.ds(0, HPT)],
                        table.at[pl.ds(s * HPT, HPT)])
        plsc.subcore_barrier()
        pltpu.sync_copy(labels_hbm.at[pl.ds(s * 8, 8)],
                        lab_v[0].at[pl.ds(0, 8)])
        for j in range(8):
            for k in range(8):
                lab = lab_v[0][j, pl.ds(k * 16, 16)]
                loc = lab - lo
                m = (loc >= 0) & (loc < HLF)
                lab_v[1][j, pl.ds(k * 16, 16)] = jnp.where(m, loc, DUMMY)
        for j in range(8):
            pltpu.sync_copy(ones_v, table.at[lab_v[1].at[j]], add=True)
        plsc.subcore_barrier()
        pltpu.sync_copy(labels_hbm.at[pl.ds(w * 4, 4)],
                        lab_v[0].at[pl.ds(0, 4)])
        for j in range(4):
            for k in range(8):
                lab = lab_v[0][j, pl.ds(k * 16, 16)]
                loc = lab - lo
                m = (loc >= 0) & (loc < HLF)
                lab_v[1][j, pl.ds(k * 16, 16)] = jnp.where(m, loc, DUMMY)
        for j in range(4):
            pltpu.sync_copy(table.at[lab_v[1].at[j]],
                            featd_v[0].at[pl.ds(j * 128, 128)])
        for j in range(4):
            for k in range(8):
                lab = lab_v[0][j, pl.ds(k * 16, 16)]
                loc = lab - lo
                m = (loc >= 0) & (loc < HLF)
                g = featd_v[0][pl.ds(j * 128 + k * 16, 16)]
                cur = num_v[pl.ds(j * 128 + k * 16, 16)]
                num_v[pl.ds(j * 128 + k * 16, 16)] = jnp.where(m, g, cur)
        plsc.subcore_barrier()
    pltpu.sync_copy(num_v, num_hbm.at[pl.ds(w * BPW, BPW)])

    def _fire(p, chunk, slot):
        d = c * 32 + p * 16 + s
        cps = (
            pltpu.async_copy(labels_hbm.at[pl.ds(chunk * 16, 16)],
                             lab_v[slot], ch_sems[slot]),
            pltpu.async_copy(featT_hbm.at[d, pl.ds(chunk * CHUNK, CHUNK)],
                             featd_v[slot], ch_sems[slot]),
        )
        return cps

    for p in range(2):
        if p == 0:
            for cp in dim_cps:
                cp.wait()
        else:
            pltpu.sync_copy(centersT_hbm.at[c * 32 + 16 + s], dimrow_v)
        cps = _fire(p, 0, 0)
        for chunk in range(NCHUNK):
            nxt = None
            if chunk + 1 < NCHUNK:
                nxt = _fire(p, chunk + 1, (chunk + 1) % 2)
            for cp in cps:
                cp.wait()
            slot = chunk % 2
            base = chunk * CHUNK

            def _step(st, _):
                r = st >> 3
                o = (st & 7) * 16
                idx = lab_v[slot][r, pl.ds(o, 16)]
                cv = plsc.load_gather(dimrow_v, [idx])
                f = featd_v[slot][pl.ds(st * 16, 16)]
                d = f - cv
                dd = d * d
                off = base + st * 16
                if p == 0:
                    partial_v[pl.ds(off, 16)] = dd
                else:
                    partial_v[pl.ds(off, 16)] = partial_v[pl.ds(off, 16)] + dd
                return ()
            lax.fori_loop(0, CHUNK // 16, _step, ())
            cps = nxt

    for r in range(16):
        pltpu.sync_copy(partial_v.at[pl.ds(r * 1024, 1024)],
                        parts_hbm.at[w * 16 + r])


@jax.jit
def _sc_stage(labels2d, featT, centersT):
    mesh = plsc.VectorSubcoreMesh(core_axis_name="c", subcore_axis_name="s")
    fn = pl.kernel(
        _sc_body,
        out_type=(
            jax.ShapeDtypeStruct((512, 1024), jnp.float32),
            jax.ShapeDtypeStruct((BATCH,), jnp.float32),
        ),
        mesh=mesh,
        compiler_params=pltpu.CompilerParams(
            needs_layout_passes=False, use_tc_tiling_on_sc=True),
        scratch_types=[
            pltpu.VMEM((NUM_CLASSES,), jnp.float32),
            pltpu.VMEM((BATCH,), jnp.float32),
            [pltpu.VMEM((16, 128), jnp.int32) for _ in range(2)],
            [pltpu.VMEM((CHUNK,), jnp.float32) for _ in range(2)],
            pltpu.VMEM((BPW,), jnp.float32),
            pltpu.VMEM((128,), jnp.float32),
            pltpu.VMEM_SHARED((HTBL,), jnp.float32),
            pltpu.SemaphoreType.DMA,
            [pltpu.SemaphoreType.DMA for _ in range(2)],
        ],
    )
    return fn(labels2d, featT, centersT)


def _loss_body(parts_ref, num_ref, out_ref):
    sumsq = jnp.zeros((16, 1024), jnp.float32)
    for w in range(NW):
        sumsq = sumsq + parts_ref[w]
    dist = jnp.sqrt(sumsq)
    loss = jnp.sum(dist / num_ref[...]) * (1.0 / BATCH)
    out_ref[...] = loss.reshape(1, 1)


@jax.jit
def _tc_stage(parts, num):
    out = pl.pallas_call(
        _loss_body,
        out_shape=jax.ShapeDtypeStruct((1, 1), jnp.float32),
    )(parts.reshape(NW, 16, 1024), num.reshape(16, 1024))
    return out[0, 0]


def kernel(feature, label, centers):
    label = jnp.asarray(label, jnp.int32)
    labels2d = label.reshape(128, 128)
    featT = jnp.transpose(feature)
    centersT = jnp.transpose(centers)
    parts, num = _sc_stage(labels2d, featT, centersT)
    return _tc_stage(parts, num)
